# Initial kernel scaffold; baseline (speedup 1.0000x reference)
import jax
import jax.numpy as jnp
from jax.experimental import pallas as pl


def kernel(x, edge_index, W1, W2):
    raise NotImplementedError("write your pallas kernel here")


# speedup vs baseline: 18.6749x; 18.6749x over previous
import functools

import jax
import jax.numpy as jnp
from jax import lax
from jax.experimental import pallas as pl
from jax.experimental.pallas import tpu as pltpu
from jax.experimental.pallas import tpu_sc as plsc

N = 10000
D = 128
DH = D // 2
E = 320000
NC = 2
NS = 16
NW = NC * NS
EPW = E // NW
K = 80
NCH = EPW // K
ZB = 624
ZREM = N - NS * ZB

_mesh = plsc.VectorSubcoreMesh(core_axis_name="c", subcore_axis_name="s")

_f32 = jnp.float32

_sc_params = pltpu.CompilerParams(use_tc_tiling_on_sc=False)


@functools.partial(
    pl.kernel,
    out_type=(jax.ShapeDtypeStruct((N,), _f32),
              jax.ShapeDtypeStruct((N,), _f32)),
    mesh=_mesh,
    scratch_types=[
        pltpu.VMEM((NCH, K), jnp.int32),
        pltpu.VMEM((K,), _f32),
        pltpu.VMEM((N,), _f32),
        pltpu.VMEM_SHARED((N,), _f32),
    ],
    compiler_params=_sc_params,
)
def _deg_kernel(dst_hbm, d0_hbm, d1_hbm, didx, ones_v, zero_v, acc):
    c = lax.axis_index("c")
    s = lax.axis_index("s")
    wid = s * NC + c

    pltpu.sync_copy(dst_hbm.at[wid], didx)

    def fill(i, _):
        ones_v[pl.ds(i * 16, 16)] = jnp.ones((16,), _f32)
        return 0
    lax.fori_loop(0, K // 16, fill, 0)

    @pl.when(s == 0)
    def _():
        def zfill(i, _):
            zero_v[pl.ds(i * 16, 16)] = jnp.zeros((16,), _f32)
            return 0
        lax.fori_loop(0, N // 16, zfill, 0)
        pltpu.sync_copy(zero_v, acc)

    plsc.subcore_barrier()

    def body(j, _):
        pltpu.sync_copy(ones_v, acc.at[didx.at[j]], add=True)
        return 0
    lax.fori_loop(0, NCH, body, 0)

    plsc.subcore_barrier()

    @pl.when((s == 0) & (c == 0))
    def _():
        pltpu.sync_copy(acc, d0_hbm)

    @pl.when((s == 0) & (c == 1))
    def _():
        pltpu.sync_copy(acc, d1_hbm)


@functools.partial(
    pl.kernel,
    out_type=tuple(jax.ShapeDtypeStruct((N, DH), _f32) for _ in range(4)),
    mesh=_mesh,
    scratch_types=[
        pltpu.VMEM((NCH, K), jnp.int32),
        pltpu.VMEM((NCH, K), jnp.int32),
        pltpu.VMEM((K, DH), _f32),
        pltpu.VMEM((K, DH), _f32),
        pltpu.VMEM((K, DH), _f32),
        pltpu.VMEM_SHARED((N, DH), _f32),
        pltpu.SemaphoreType.DMA,
        pltpu.SemaphoreType.DMA,
    ],
    compiler_params=_sc_params,
)
def _agg_kernel(tlo_hbm, thi_hbm, src_hbm, dst_hbm,
                p0lo_hbm, p1lo_hbm, p0hi_hbm, p1hi_hbm,
                sidx, didx, buf_a, buf_b, buf_z, acc, sem_a, sem_b):
    c = lax.axis_index("c")
    s = lax.axis_index("s")
    wid = s * NC + c
    base = s * ZB

    pltpu.sync_copy(src_hbm.at[wid], sidx)
    pltpu.sync_copy(dst_hbm.at[wid], didx)

    zv = jnp.zeros((16,), _f32)

    def zrow(r, _):
        for q in range(DH // 16):
            buf_z[r, pl.ds(q * 16, 16)] = zv
        return 0
    lax.fori_loop(0, K, zrow, 0)

    def zero_acc_slice():
        for z in range(ZB // K):
            pltpu.sync_copy(buf_z, acc.at[pl.ds(base + z * K, K)])
        rem = ZB % K
        pltpu.sync_copy(buf_z.at[pl.ds(0, rem)],
                        acc.at[pl.ds(base + (ZB // K) * K, rem)])

        @pl.when(s == 0)
        def _():
            pltpu.sync_copy(buf_z.at[pl.ds(0, ZREM)],
                            acc.at[pl.ds(NS * ZB, ZREM)])

    zero_acc_slice()
    plsc.subcore_barrier()

    for table_hbm, pc0_hbm, pc1_hbm, last in (
        (tlo_hbm, p0lo_hbm, p1lo_hbm, False),
        (thi_hbm, p0hi_hbm, p1hi_hbm, True),
    ):
        pltpu.async_copy(table_hbm.at[sidx.at[0]], buf_a, sem_a)

        def body(i, _):
            j0 = 2 * i
            pltpu.async_copy(table_hbm.at[sidx.at[j0 + 1]], buf_b, sem_b)
            pltpu.make_async_copy(table_hbm.at[sidx.at[j0]], buf_a, sem_a).wait()
            pltpu.sync_copy(buf_a, acc.at[didx.at[j0]], add=True)
            pltpu.async_copy(table_hbm.at[sidx.at[j0 + 2]], buf_a, sem_a)
            pltpu.make_async_copy(table_hbm.at[sidx.at[j0 + 1]], buf_b, sem_b).wait()
            pltpu.sync_copy(buf_b, acc.at[didx.at[j0 + 1]], add=True)
            return 0
        lax.fori_loop(0, (NCH - 1) // 2, body, 0)

        pltpu.make_async_copy(table_hbm.at[sidx.at[NCH - 1]], buf_a, sem_a).wait()
        pltpu.sync_copy(buf_a, acc.at[didx.at[NCH - 1]], add=True)

        plsc.subcore_barrier()

        @pl.when(c == 0)
        def _():
            pltpu.sync_copy(acc.at[pl.ds(base, ZB)],
                            pc0_hbm.at[pl.ds(base, ZB)])

        @pl.when(c == 1)
        def _():
            pltpu.sync_copy(acc.at[pl.ds(base, ZB)],
                            pc1_hbm.at[pl.ds(base, ZB)])

        @pl.when((s == 0) & (c == 0))
        def _():
            pltpu.sync_copy(acc.at[pl.ds(NS * ZB, ZREM)],
                            pc0_hbm.at[pl.ds(NS * ZB, ZREM)])

        @pl.when((s == 0) & (c == 1))
        def _():
            pltpu.sync_copy(acc.at[pl.ds(NS * ZB, ZREM)],
                            pc1_hbm.at[pl.ds(NS * ZB, ZREM)])

        if not last:
            zero_acc_slice()
            plsc.subcore_barrier()


_RB = 1000


def _prep_body(x_ref, d0_ref, d1_ref, xlo_ref, xhi_ref, dinv_ref):
    deg = jnp.maximum(d0_ref[...] + d1_ref[...], 1.0)
    di = lax.rsqrt(deg)
    dinv_ref[...] = di
    xs = x_ref[...] * di
    xlo_ref[...] = xs[:, :DH]
    xhi_ref[...] = xs[:, DH:]


_prep_call = pl.pallas_call(
    _prep_body,
    grid=(N // _RB,),
    in_specs=[
        pl.BlockSpec((_RB, D), lambda i: (i, 0)),
        pl.BlockSpec((_RB, 1), lambda i: (i, 0)),
        pl.BlockSpec((_RB, 1), lambda i: (i, 0)),
    ],
    out_specs=[
        pl.BlockSpec((_RB, DH), lambda i: (i, 0)),
        pl.BlockSpec((_RB, DH), lambda i: (i, 0)),
        pl.BlockSpec((_RB, 1), lambda i: (i, 0)),
    ],
    out_shape=[
        jax.ShapeDtypeStruct((N, DH), _f32),
        jax.ShapeDtypeStruct((N, DH), _f32),
        jax.ShapeDtypeStruct((N, 1), _f32),
    ],
)


def _dense_body(p0lo_ref, p1lo_ref, p0hi_ref, p1hi_ref, dinv_ref, w_ref,
                *out_refs, relu_scale):
    di = dinv_ref[...]
    alo = (p0lo_ref[...] + p1lo_ref[...]) * di
    ahi = (p0hi_ref[...] + p1hi_ref[...]) * di
    w = w_ref[...]
    m = (jnp.dot(alo, w[:DH, :], preferred_element_type=_f32)
         + jnp.dot(ahi, w[DH:, :], preferred_element_type=_f32))
    if relu_scale:
        v = jnp.maximum(m, 0.0) * di
        out_refs[0][...] = v[:, :DH]
        out_refs[1][...] = v[:, DH:]
    else:
        out_refs[0][...] = m


def _make_dense(relu_scale):
    if relu_scale:
        out_specs = [pl.BlockSpec((_RB, DH), lambda i: (i, 0)),
                     pl.BlockSpec((_RB, DH), lambda i: (i, 0))]
        out_shape = [jax.ShapeDtypeStruct((N, DH), _f32),
                     jax.ShapeDtypeStruct((N, DH), _f32)]
    else:
        out_specs = pl.BlockSpec((_RB, D), lambda i: (i, 0))
        out_shape = jax.ShapeDtypeStruct((N, D), _f32)
    return pl.pallas_call(
        functools.partial(_dense_body, relu_scale=relu_scale),
        grid=(N // _RB,),
        in_specs=[
            pl.BlockSpec((_RB, DH), lambda i: (i, 0)),
            pl.BlockSpec((_RB, DH), lambda i: (i, 0)),
            pl.BlockSpec((_RB, DH), lambda i: (i, 0)),
            pl.BlockSpec((_RB, DH), lambda i: (i, 0)),
            pl.BlockSpec((_RB, 1), lambda i: (i, 0)),
            pl.BlockSpec((D, D), lambda i: (0, 0)),
        ],
        out_specs=out_specs,
        out_shape=out_shape,
    )


_dense_relu = _make_dense(True)
_dense_plain = _make_dense(False)


def kernel(x, edge_index, W1, W2):
    src = edge_index[0].reshape(NW, NCH, K)
    dst = edge_index[1].reshape(NW, NCH, K)

    d0, d1 = _deg_kernel(dst)
    xlo, xhi, dinv = _prep_call(x, d0.reshape(N, 1), d1.reshape(N, 1))

    p0lo, p1lo, p0hi, p1hi = _agg_kernel(xlo, xhi, src, dst)
    h1lo, h1hi = _dense_relu(p0lo, p1lo, p0hi, p1hi, dinv, W1)

    q0lo, q1lo, q0hi, q1hi = _agg_kernel(h1lo, h1hi, src, dst)
    h2 = _dense_plain(q0lo, q1lo, q0hi, q1hi, dinv, W2)
    return h2
